# baseline (speedup 1.0000x reference)
import functools

import jax
import jax.numpy as jnp
from jax import lax
from jax.experimental import pallas as pl
from jax.experimental.pallas import tpu as pltpu
from jax.experimental.pallas import tpu_sc as plsc

DIM = 32
NUM_WORKERS = 32
CHUNK = 512
NBUF = 2
STRIPE = 896
TAIL = 64

_MESH = plsc.VectorSubcoreMesh(core_axis_name="c", subcore_axis_name="s")
_PARAMS = pltpu.CompilerParams(use_tc_tiling_on_sc=False)
_PARAMS_NOLAYOUT = pltpu.CompilerParams(use_tc_tiling_on_sc=False,
                                        needs_layout_passes=False)


def _transpose_body(vin_ref, vout_ref, n_groups):
    iota = jax.lax.iota(jnp.int32, 16)

    def group(g, carry):
        addr0 = carry
        base = g * 16
        for f in range(DIM):
            x = vin_ref[f, pl.ds(base, 16)]
            plsc.store_scatter(vout_ref, [addr0 + f], x)
        return addr0 + 16 * DIM

    lax.fori_loop(0, n_groups, group, iota * DIM, unroll=False)


def _transpose_tables(t0t, t1t, t0tail, t1tail, vocab):
    n_stripes = (vocab - TAIL) // STRIPE
    assert (vocab - TAIL) % STRIPE == 0 and STRIPE % 16 == 0
    vrows = STRIPE // 4
    trows = TAIL // 4
    n_iter = (n_stripes + NUM_WORKERS - 1) // NUM_WORKERS

    @functools.partial(
        pl.kernel,
        mesh=_MESH,
        compiler_params=pltpu.CompilerParams(needs_layout_passes=False,
                                             disable_bounds_checks=True),
        out_type=(
            jax.ShapeDtypeStruct((vocab * DIM,), jnp.float32),
            jax.ShapeDtypeStruct((vocab * DIM,), jnp.float32),
        ),
        scratch_types=[
            pltpu.VMEM((DIM, STRIPE + 1), jnp.float32),
            pltpu.VMEM((DIM, STRIPE + 1), jnp.float32),
            pltpu.VMEM((STRIPE * DIM,), jnp.float32),
            pltpu.VMEM((STRIPE * DIM,), jnp.float32),
            pltpu.VMEM((DIM, TAIL), jnp.float32),
            pltpu.VMEM((TAIL * DIM,), jnp.float32),
            pltpu.SemaphoreType.DMA,
            pltpu.SemaphoreType.DMA,
            pltpu.SemaphoreType.DMA,
            pltpu.SemaphoreType.DMA,
        ],
    )
    def k(t0_hbm, t1_hbm, tl0_hbm, tl1_hbm, s0_hbm, s1_hbm,
          vin0, vin1, vout0, vout1, vtin, vtout, rs0, rs1, ws0, ws1):
        wid = lax.axis_index("s") * 2 + lax.axis_index("c")
        vins, vouts, rsems, wsems = (vin0, vin1), (vout0, vout1), \
            (rs0, rs1), (ws0, ws1)

        def do_table(src_hbm, dst_hbm):
            def fire_read(it, b):
                sid = wid + it * NUM_WORKERS

                @pl.when(sid < n_stripes)
                def _():
                    pltpu.async_copy(
                        src_hbm.at[:, pl.ds(sid * STRIPE, STRIPE)],
                        vins[b].at[:, pl.ds(0, STRIPE)], rsems[b])

            def wait_read(b):
                pltpu.make_async_copy(
                    src_hbm.at[:, pl.ds(0, STRIPE)],
                    vins[b].at[:, pl.ds(0, STRIPE)], rsems[b]).wait()

            def wait_write(b):
                pltpu.make_async_copy(
                    vouts[b], dst_hbm.at[pl.ds(0, STRIPE * DIM)],
                    wsems[b]).wait()

            fire_read(0, 0)
            fire_read(1, 1)

            def body(it, carry):
                for b in range(2):
                    j = it * 2 + b
                    sid = wid + j * NUM_WORKERS

                    @pl.when(sid < n_stripes)
                    def _():
                        wait_read(b)

                        @pl.when(j >= 2)
                        def _():
                            wait_write(b)

                        _transpose_body(vins[b], vouts[b], STRIPE // 16)
                        pltpu.async_copy(
                            vouts[b],
                            dst_hbm.at[pl.ds(sid * STRIPE * DIM,
                                             STRIPE * DIM)],
                            wsems[b])
                        fire_read(j + 2, b)
                return carry

            lax.fori_loop(0, (n_iter + 1) // 2, body, 0, unroll=False)
            for j in range(max(0, n_iter - 3), n_iter):
                fired = wid + j * NUM_WORKERS < n_stripes
                fired_n2 = wid + (j + 2) * NUM_WORKERS < n_stripes

                @pl.when(jnp.logical_and(fired, jnp.logical_not(fired_n2)))
                def _():
                    wait_write(j % 2)

        do_table(t0_hbm, s0_hbm)
        do_table(t1_hbm, s1_hbm)

        @pl.when(wid == 0)
        def _():
            for tl_hbm, dst_hbm in ((tl0_hbm, s0_hbm), (tl1_hbm, s1_hbm)):
                pltpu.sync_copy(tl_hbm, vtin)
                _transpose_body(vtin, vtout, TAIL // 16)
                pltpu.sync_copy(
                    vtout,
                    dst_hbm.at[pl.ds((vocab - TAIL) * DIM, TAIL * DIM)])

    return k(t0t, t1t, t0tail, t1tail)


@functools.partial(jax.jit, static_argnums=(3, 4))
def _combine(idx_flat, table0, table1, total, per_worker):
    n_chunks = per_worker // CHUNK
    assert per_worker % CHUNK == 0 and n_chunks % NBUF == 0

    row_bufs = [
        [pltpu.VMEM((CHUNK, DIM), jnp.float32) for _ in range(2)]
        for _ in range(NBUF)
    ]
    gather_sems = [pltpu.SemaphoreType.DMA for _ in range(NBUF)]
    write_sems = [pltpu.SemaphoreType.DMA for _ in range(NBUF)]

    @functools.partial(
        pl.kernel,
        mesh=_MESH,
        compiler_params=_PARAMS,
        out_type=jax.ShapeDtypeStruct((total, 2 * DIM), jnp.float32),
        scratch_types=[pltpu.VMEM((per_worker,), jnp.int32), row_bufs,
                       gather_sems, write_sems],
    )
    def k(idx_hbm, t0_hbm, t1_hbm, out_hbm, idx_v, rbufs, gsems, wsems):
        wid = lax.axis_index("s") * 2 + lax.axis_index("c")
        base_w = wid * per_worker
        pltpu.sync_copy(idx_hbm.at[pl.ds(base_w, per_worker)], idx_v)

        def fire_gathers(i, b):
            sl = idx_v.at[pl.ds(i * CHUNK, CHUNK)]
            pltpu.async_copy(t0_hbm.at[sl], rbufs[b][0], gsems[b])
            pltpu.async_copy(t1_hbm.at[sl], rbufs[b][1], gsems[b])

        def wait_gathers(i, b):
            pltpu.make_async_copy(t0_hbm.at[idx_v.at[pl.ds(0, CHUNK)]],
                                  rbufs[b][0], gsems[b]).wait()
            pltpu.make_async_copy(t1_hbm.at[idx_v.at[pl.ds(0, CHUNK)]],
                                  rbufs[b][1], gsems[b]).wait()

        def fire_writes(i, b):
            base = base_w + i * CHUNK
            pltpu.async_copy(rbufs[b][0],
                             out_hbm.at[pl.ds(base, CHUNK), pl.ds(0, DIM)],
                             wsems[b])
            pltpu.async_copy(rbufs[b][1],
                             out_hbm.at[pl.ds(base, CHUNK), pl.ds(DIM, DIM)],
                             wsems[b])

        def wait_writes(b):
            pltpu.make_async_copy(rbufs[b][0],
                                  out_hbm.at[pl.ds(0, CHUNK), pl.ds(0, DIM)],
                                  wsems[b]).wait()
            pltpu.make_async_copy(rbufs[b][1],
                                  out_hbm.at[pl.ds(0, CHUNK), pl.ds(DIM, DIM)],
                                  wsems[b]).wait()

        for b in range(NBUF):
            fire_gathers(b, b)

        def body(g, carry):
            for b in range(NBUF):
                i = g + b
                wait_gathers(i, b)
                fire_writes(i, b)
                wait_writes(b)
                fire_gathers(i + NBUF, b)
            return carry

        lax.fori_loop(0, (n_chunks - NBUF) // NBUF,
                      lambda t, c: body(t * NBUF, c), 0, unroll=False)

        g0 = n_chunks - NBUF
        for b in range(NBUF):
            i = g0 + b
            wait_gathers(i, b)
            fire_writes(i, b)
            wait_writes(b)

    return k(idx_flat, table0, table1)


def kernel(input, table0, table1):
    B, L = input.shape
    total = B * L
    vocab = table0.shape[0]
    idx_flat = input.T.reshape(total).astype(jnp.int32)
    s0f, s1f = _transpose_tables(table0.T, table1.T,
                                 table0[vocab - TAIL:].T,
                                 table1[vocab - TAIL:].T, vocab)
    s0 = s0f.reshape(vocab, DIM)
    s1 = s1f.reshape(vocab, DIM)
    per_worker = total // NUM_WORKERS
    out = _combine(idx_flat, s0, s1, total, per_worker)
    return out.reshape(L, B, 2 * DIM).transpose(1, 0, 2)
